# baseline (speedup 1.0000x reference)
import functools

import jax
import jax.numpy as jnp
from jax import lax
from jax.experimental import pallas as pl
from jax.experimental.pallas import tpu as pltpu
from jax.experimental.pallas import tpu_sc as plsc

B = 16384
D = 64
NC = 2
NS = 16
NW = NC * NS
BPW = B // NW
CHUNK = 128
NCHUNK = BPW // CHUNK
L = 16


def _mf_body(u_idx_hbm, i_idx_hbm, u_emb_hbm, i_emb_hbm, u_bias_hbm,
             i_bias_hbm, out_hbm,
             uidx_v, iidx_v, urows, irows, ub_v, ib_v, out_v, sem):
    wid = lax.axis_index("s") * NC + lax.axis_index("c")
    base = wid * BPW

    pltpu.sync_copy(u_idx_hbm.at[wid], uidx_v)
    pltpu.sync_copy(i_idx_hbm.at[wid], iidx_v)

    copies = []
    for k in range(NCHUNK):
        sl = pl.ds(k * CHUNK, CHUNK)
        copies.append(pltpu.async_copy(u_emb_hbm.at[uidx_v.at[k]], urows.at[sl], sem))
        copies.append(pltpu.async_copy(i_emb_hbm.at[iidx_v.at[k]], irows.at[sl], sem))
        copies.append(pltpu.async_copy(u_bias_hbm.at[uidx_v.at[k]], ub_v.at[sl], sem))
        copies.append(pltpu.async_copy(i_bias_hbm.at[iidx_v.at[k]], ib_v.at[sl], sem))
    for c in copies:
        c.wait()

    lane = lax.broadcasted_iota(jnp.int32, (L,), 0)
    onehots = [(lane == j).astype(jnp.float32) for j in range(L)]

    def group(g, carry):
        sl = pl.ds(g * L, L)
        accv = ub_v[sl] + ib_v[sl]
        for j in range(L):
            r = g * L + j
            prod = urows[r, pl.ds(0, L)] * irows[r, pl.ds(0, L)]
            for c in range(1, D // L):
                prod = prod + urows[r, pl.ds(c * L, L)] * irows[r, pl.ds(c * L, L)]
            s = jnp.sum(prod)
            accv = accv + jnp.broadcast_to(s, (L,)) * onehots[j]
        out_v[sl] = accv
        return carry

    lax.fori_loop(0, BPW // L, group, 0)

    pltpu.sync_copy(out_v, out_hbm.at[pl.ds(base, BPW)])


@functools.partial(jax.jit, static_argnums=())
def kernel(u_idx, i_idx, u_emb, i_emb, u_bias, i_bias):
    mesh = plsc.VectorSubcoreMesh(core_axis_name="c", subcore_axis_name="s")
    run = functools.partial(
        pl.kernel,
        mesh=mesh,
        compiler_params=pltpu.CompilerParams(
            needs_layout_passes=False, use_tc_tiling_on_sc=False),
        out_type=jax.ShapeDtypeStruct((B,), jnp.float32),
        scratch_types=[
            pltpu.VMEM((NCHUNK, CHUNK), jnp.int32),
            pltpu.VMEM((NCHUNK, CHUNK), jnp.int32),
            pltpu.VMEM((BPW, D), jnp.float32),
            pltpu.VMEM((BPW, D), jnp.float32),
            pltpu.VMEM((BPW,), jnp.float32),
            pltpu.VMEM((BPW,), jnp.float32),
            pltpu.VMEM((BPW,), jnp.float32),
            pltpu.SemaphoreType.DMA,
        ],
    )(_mf_body)
    u_idx_r = u_idx.reshape(NW, NCHUNK, CHUNK)
    i_idx_r = i_idx.reshape(NW, NCHUNK, CHUNK)
    return run(u_idx_r, i_idx_r, u_emb, i_emb,
               u_bias.reshape(-1), i_bias.reshape(-1))
